# baseline (speedup 1.0000x reference)
import functools

import jax
import jax.numpy as jnp
from jax.experimental import pallas as pl

HIDDEN = 4096
NUM_EXPERTS = 64
TOP_K = 8
NEG_INF = float("-inf")


def _router_body(x_ref, wt_ref, b_ref, logits_ref, w_ref, i_ref):
    logits = (
        jnp.dot(x_ref[...], wt_ref[...], preferred_element_type=jnp.float32)
        + b_ref[...]
    )
    logits_ref[...] = logits

    lane = jax.lax.broadcasted_iota(jnp.int32, logits.shape, 1)
    work = logits
    vals = []
    idxs = []
    for _ in range(TOP_K):
        m = jnp.max(work, axis=1, keepdims=True)
        idx = jnp.min(
            jnp.where(work == m, lane, NUM_EXPERTS), axis=1, keepdims=True
        )
        vals.append(m)
        idxs.append(idx)
        work = jnp.where(lane == idx, NEG_INF, work)
    v = jnp.concatenate(vals, axis=1)
    i = jnp.concatenate(idxs, axis=1)

    e = jnp.exp(v - v[:, 0:1])
    w_ref[...] = e / jnp.sum(e, axis=1, keepdims=True)
    i_ref[...] = i


@functools.partial(jax.jit, static_argnames=("block_tokens",))
def _router(hidden_states, W, b, block_tokens=512):
    B, S, H = hidden_states.shape
    T = B * S
    x = hidden_states.reshape(T, H)
    wt = W.T
    b2 = b.reshape(1, NUM_EXPERTS)

    grid = (T // block_tokens,)
    logits, weights, indices = pl.pallas_call(
        _router_body,
        grid=grid,
        in_specs=[
            pl.BlockSpec((block_tokens, H), lambda t: (t, 0)),
            pl.BlockSpec((H, NUM_EXPERTS), lambda t: (0, 0)),
            pl.BlockSpec((1, NUM_EXPERTS), lambda t: (0, 0)),
        ],
        out_specs=[
            pl.BlockSpec((block_tokens, NUM_EXPERTS), lambda t: (t, 0)),
            pl.BlockSpec((block_tokens, TOP_K), lambda t: (t, 0)),
            pl.BlockSpec((block_tokens, TOP_K), lambda t: (t, 0)),
        ],
        out_shape=[
            jax.ShapeDtypeStruct((T, NUM_EXPERTS), jnp.float32),
            jax.ShapeDtypeStruct((T, TOP_K), jnp.float32),
            jax.ShapeDtypeStruct((T, TOP_K), jnp.int32),
        ],
    )(x, wt, b2)

    return (
        weights.reshape(B, S, TOP_K),
        indices.reshape(B, S, TOP_K),
        logits.reshape(B, S, NUM_EXPERTS),
    )


def kernel(hidden_states, W, b):
    return _router(hidden_states, W, b)


# speedup vs baseline: 1.3896x; 1.3896x over previous
import functools

import jax
import jax.numpy as jnp
from jax.experimental import pallas as pl

HIDDEN = 4096
NUM_EXPERTS = 64
TOP_K = 8
NEG_INF = float("-inf")


def _router_body(x_ref, wt_ref, b_ref, logits_ref, w_ref, i_ref):
    logits = (
        jnp.dot(x_ref[...], wt_ref[...], preferred_element_type=jnp.float32)
        + b_ref[...]
    )
    logits_ref[...] = logits

    work = logits.T
    eid = jax.lax.broadcasted_iota(jnp.int32, work.shape, 0).astype(jnp.float32)
    vals = []
    idxs = []
    for _ in range(TOP_K):
        m = jnp.max(work, axis=0, keepdims=True)
        idx = jnp.min(
            jnp.where(work == m, eid, float(NUM_EXPERTS)), axis=0, keepdims=True
        )
        vals.append(m)
        idxs.append(idx)
        work = jnp.where(eid == idx, NEG_INF, work)
    v = jnp.concatenate(vals, axis=0)
    i = jnp.concatenate(idxs, axis=0)

    e = jnp.exp(v - v[0:1, :])
    w = e / jnp.sum(e, axis=0, keepdims=True)
    w_ref[...] = w.T
    i_ref[...] = i.T.astype(jnp.int32)


@functools.partial(jax.jit, static_argnames=("block_tokens",))
def _router(hidden_states, W, b, block_tokens=512):
    B, S, H = hidden_states.shape
    T = B * S
    x = hidden_states.reshape(T, H)
    wt = W.T
    b2 = b.reshape(1, NUM_EXPERTS)

    grid = (T // block_tokens,)
    logits, weights, indices = pl.pallas_call(
        _router_body,
        grid=grid,
        in_specs=[
            pl.BlockSpec((block_tokens, H), lambda t: (t, 0)),
            pl.BlockSpec((H, NUM_EXPERTS), lambda t: (0, 0)),
            pl.BlockSpec((1, NUM_EXPERTS), lambda t: (0, 0)),
        ],
        out_specs=[
            pl.BlockSpec((block_tokens, NUM_EXPERTS), lambda t: (t, 0)),
            pl.BlockSpec((block_tokens, TOP_K), lambda t: (t, 0)),
            pl.BlockSpec((block_tokens, TOP_K), lambda t: (t, 0)),
        ],
        out_shape=[
            jax.ShapeDtypeStruct((T, NUM_EXPERTS), jnp.float32),
            jax.ShapeDtypeStruct((T, TOP_K), jnp.float32),
            jax.ShapeDtypeStruct((T, TOP_K), jnp.int32),
        ],
    )(x, wt, b2)

    return (
        weights.reshape(B, S, TOP_K),
        indices.reshape(B, S, TOP_K),
        logits.reshape(B, S, NUM_EXPERTS),
    )


def kernel(hidden_states, W, b):
    return _router(hidden_states, W, b)


# speedup vs baseline: 1.4422x; 1.0378x over previous
import functools

import jax
import jax.numpy as jnp
from jax.experimental import pallas as pl

HIDDEN = 4096
NUM_EXPERTS = 64
TOP_K = 8
NEG_INF = float("-inf")


def _router_body(x_ref, wt_ref, b_ref, logits_ref, w_ref, i_ref):
    logits = (
        jnp.dot(x_ref[...], wt_ref[...], preferred_element_type=jnp.float32)
        + b_ref[...]
    )
    logits_ref[...] = logits

    work = logits.T
    eid = jax.lax.broadcasted_iota(jnp.int32, work.shape, 0).astype(jnp.float32)
    vals = []
    idxs = []
    for _ in range(TOP_K):
        m = jnp.max(work, axis=0, keepdims=True)
        idx = jnp.min(
            jnp.where(work == m, eid, float(NUM_EXPERTS)), axis=0, keepdims=True
        )
        vals.append(m)
        idxs.append(idx)
        work = jnp.where(eid == idx, NEG_INF, work)
    v = jnp.concatenate(vals, axis=0)
    i = jnp.concatenate(idxs, axis=0)

    e = jnp.exp(v - v[0:1, :])
    w = e / jnp.sum(e, axis=0, keepdims=True)
    w_ref[...] = w.T
    i_ref[...] = i.T.astype(jnp.int32)


@functools.partial(jax.jit, static_argnames=("block_tokens",))
def _router(hidden_states, W, b, block_tokens=1024):
    B, S, H = hidden_states.shape
    T = B * S
    x = hidden_states.reshape(T, H)
    wt = W.T
    b2 = b.reshape(1, NUM_EXPERTS)

    grid = (T // block_tokens,)
    logits, weights, indices = pl.pallas_call(
        _router_body,
        grid=grid,
        in_specs=[
            pl.BlockSpec((block_tokens, H), lambda t: (t, 0)),
            pl.BlockSpec((H, NUM_EXPERTS), lambda t: (0, 0)),
            pl.BlockSpec((1, NUM_EXPERTS), lambda t: (0, 0)),
        ],
        out_specs=[
            pl.BlockSpec((block_tokens, NUM_EXPERTS), lambda t: (t, 0)),
            pl.BlockSpec((block_tokens, TOP_K), lambda t: (t, 0)),
            pl.BlockSpec((block_tokens, TOP_K), lambda t: (t, 0)),
        ],
        out_shape=[
            jax.ShapeDtypeStruct((T, NUM_EXPERTS), jnp.float32),
            jax.ShapeDtypeStruct((T, TOP_K), jnp.float32),
            jax.ShapeDtypeStruct((T, TOP_K), jnp.int32),
        ],
    )(x, wt, b2)

    return (
        weights.reshape(B, S, TOP_K),
        indices.reshape(B, S, TOP_K),
        logits.reshape(B, S, NUM_EXPERTS),
    )


def kernel(hidden_states, W, b):
    return _router(hidden_states, W, b)
